# Initial kernel scaffold; baseline (speedup 1.0000x reference)
import jax
import jax.numpy as jnp
from jax.experimental import pallas as pl


def kernel(x, emb_table, pos_table):
    raise NotImplementedError("write your pallas kernel here")


# speedup vs baseline: 1.1179x; 1.1179x over previous
import functools

import jax
import jax.numpy as jnp
from jax import lax
from jax.experimental import pallas as pl
from jax.experimental.pallas import tpu as pltpu
from jax.experimental.pallas import tpu_sc as plsc

NUM_VOCAB = 1000000
MAXLEN = 200
H = 32
CH = 100
BLK = 16
NC = 2
NS = 16
NW = NC * NS


def _body(x_hbm, emb_hbm, pos_hbm, out_hbm, idx_v, rows_v, pos_v, sem):
    cid = lax.axis_index("c")
    sid = lax.axis_index("s")
    wid = sid * NC + cid

    n_chunks_total = out_hbm.shape[0]
    chunks_per_worker = n_chunks_total // NW
    n_blocks = chunks_per_worker // BLK

    pltpu.sync_copy(pos_hbm, pos_v)

    def block(i, carry):
        base = wid * chunks_per_worker + i * BLK
        pltpu.sync_copy(x_hbm.at[pl.ds(base, BLK)], idx_v)
        copies = [
            pltpu.async_copy(emb_hbm.at[idx_v.at[j]], rows_v.at[j], sem)
            for j in range(BLK)
        ]
        for cp in copies:
            cp.wait()

        for phase in range(2):
            def sbody(t, c, phase=phase):
                p0 = pos_v[phase * CH + t, pl.ds(0, 16)]
                p1 = pos_v[phase * CH + t, pl.ds(16, 16)]
                for j in range(phase, BLK, 2):
                    rows_v[j, t, pl.ds(0, 16)] = rows_v[j, t, pl.ds(0, 16)] + p0
                    rows_v[j, t, pl.ds(16, 16)] = rows_v[j, t, pl.ds(16, 16)] + p1
                return c
            lax.fori_loop(0, CH, sbody, 0)

        pltpu.sync_copy(rows_v, out_hbm.at[pl.ds(base, BLK)])
        return carry

    lax.fori_loop(0, n_blocks, block, 0)


def kernel(x, emb_table, pos_table):
    batch, seq_len = x.shape
    hid = emb_table.shape[1]
    n_chunks = batch * seq_len // CH
    x2 = x.reshape(n_chunks, CH).astype(jnp.int32)

    call = pl.kernel(
        _body,
        out_type=jax.ShapeDtypeStruct((n_chunks, CH, hid), jnp.float32),
        mesh=plsc.VectorSubcoreMesh(core_axis_name="c", subcore_axis_name="s"),
        scratch_types=[
            pltpu.VMEM((BLK, CH), jnp.int32),
            pltpu.VMEM((BLK, CH, hid), jnp.float32),
            pltpu.VMEM((MAXLEN, hid), jnp.float32),
            pltpu.SemaphoreType.DMA,
        ],
        compiler_params=pltpu.CompilerParams(use_tc_tiling_on_sc=False),
    )
    out = call(x2, emb_table, pos_table)
    return out.reshape(batch, seq_len, hid)


# speedup vs baseline: 1.4278x; 1.2771x over previous
import functools

import jax
import jax.numpy as jnp
from jax import lax
from jax.experimental import pallas as pl
from jax.experimental.pallas import tpu as pltpu
from jax.experimental.pallas import tpu_sc as plsc

MAXLEN = 200
H = 32
NB = 8
CH0 = 104
NC = 2
NS = 16
NW = NC * NS


def _body(x_hbm, emb_hbm, pos_hbm, out_hbm, idx_v, rows_v, pos_v, sem):
    cid = lax.axis_index("c")
    sid = lax.axis_index("s")
    wid = sid * NC + cid

    batch = out_hbm.shape[0]
    seq = out_hbm.shape[1]
    ch1 = seq - CH0
    rows_per_worker = batch // NW
    n_blocks = rows_per_worker // NB

    pltpu.sync_copy(pos_hbm, pos_v)

    def block(i, carry):
        b0 = wid * rows_per_worker + i * NB
        pltpu.sync_copy(x_hbm.at[pl.ds(b0, NB)], idx_v)
        copies = []
        for k in range(NB):
            copies.append(pltpu.async_copy(
                emb_hbm.at[idx_v.at[k, pl.ds(0, CH0)]],
                rows_v.at[k, pl.ds(0, CH0)], sem))
            copies.append(pltpu.async_copy(
                emb_hbm.at[idx_v.at[k, pl.ds(CH0, ch1)]],
                rows_v.at[k, pl.ds(CH0, ch1)], sem))
        for cp in copies:
            cp.wait()

        def sbody(t, c):
            p0 = pos_v[t, pl.ds(0, 16)]
            p1 = pos_v[t, pl.ds(16, 16)]
            for k in range(NB):
                rows_v[k, t, pl.ds(0, 16)] = rows_v[k, t, pl.ds(0, 16)] + p0
                rows_v[k, t, pl.ds(16, 16)] = rows_v[k, t, pl.ds(16, 16)] + p1
            return c
        lax.fori_loop(0, seq, sbody, 0)

        pltpu.sync_copy(rows_v, out_hbm.at[pl.ds(b0, NB)])
        return carry

    lax.fori_loop(0, n_blocks, block, 0)


def kernel(x, emb_table, pos_table):
    batch, seq_len = x.shape
    hid = emb_table.shape[1]

    call = pl.kernel(
        _body,
        out_type=jax.ShapeDtypeStruct((batch, seq_len, hid), jnp.float32),
        mesh=plsc.VectorSubcoreMesh(core_axis_name="c", subcore_axis_name="s"),
        scratch_types=[
            pltpu.VMEM((NB, seq_len), jnp.int32),
            pltpu.VMEM((NB, seq_len, hid), jnp.float32),
            pltpu.VMEM((MAXLEN, hid), jnp.float32),
            pltpu.SemaphoreType.DMA,
        ],
        compiler_params=pltpu.CompilerParams(use_tc_tiling_on_sc=False),
    )
    return call(x, emb_table, pos_table)
